# baseline (speedup 1.0000x reference)
import math

import jax
import jax.numpy as jnp
from jax import lax
from jax.experimental import pallas as pl
from jax.experimental.pallas import tpu as pltpu
from jax.experimental.pallas import tpu_sc as plsc

VOCAB = 1000000
D = 64
ROWS = 4096
COLS = 200
NC = 2
NS = 16
NW = NC * NS
TPW = ROWS // NW
L = 16
SCALE = math.sqrt(D)

_mesh = plsc.VectorSubcoreMesh(
    core_axis_name="c", subcore_axis_name="s", num_cores=NC, num_subcores=NS
)


def _body(x_hbm, tab_hbm, out_hbm, xq_v, in_v, out_v, gsem, ssem):
    wid = lax.axis_index("s") * NC + lax.axis_index("c")
    a0 = wid * TPW
    pltpu.sync_copy(x_hbm.at[:, pl.ds(a0, TPW)], xq_v)

    def start_gather(s, b):
        pltpu.async_copy(
            tab_hbm.at[xq_v.at[s]], in_v.at[b], gsem.at[b])

    def wait_gather(s, b):
        pltpu.make_async_copy(
            tab_hbm.at[xq_v.at[s]], in_v.at[b], gsem.at[b]).wait()

    def start_store(s, b):
        pltpu.async_copy(out_v.at[b], out_hbm.at[s, pl.ds(0, D // 8), wid], ssem.at[b])

    def wait_store(s, b):
        pltpu.make_async_copy(
            out_v.at[b], out_hbm.at[s, pl.ds(0, D // 8), wid], ssem.at[b]).wait()

    def transpose_add(s, b):
        lanes = lax.broadcasted_iota(jnp.int32, (L,), 0)

        @plsc.parallel_loop(0, (TPW // L) * (D // L), unroll=2)
        def _chunk(i):
            p = i >> 2
            c16 = (i & 3) * L
            rows = lanes + p * L
            for k in range(L):
                cc = ((lanes + k) & (L - 1)) + c16
                vals = plsc.load_gather(in_v.at[b], [rows, cc])
                plsc.store_scatter(
                    out_v.at[b], [cc >> 3, cc & 7, rows], vals + SCALE)


    for b in range(2):
        start_gather(b, b)

    def main(i, _):
        s0 = i * 4
        for j in range(4):
            s = s0 + j

            @pl.when(s >= 2)
            def _():
                start_store(s - 2, (j + 2) % 4)

            @pl.when(s + 2 < COLS)
            def _():
                start_gather(s + 2, (j + 2) % 4)

            wait_gather(s, j)

            @pl.when(s >= 4)
            def _():
                wait_store(s - 4, j)

            transpose_add(s, j)
        return 0

    lax.fori_loop(0, COLS // 4, main, 0)

    for s in range(COLS - 2, COLS):
        start_store(s, s % 4)
    for s in range(COLS - 4, COLS):
        wait_store(s, s % 4)


@jax.jit
def _embed(xt, tab):
    k = pl.kernel(
        _body,
        out_type=jax.ShapeDtypeStruct((COLS, D // 8, ROWS // TPW, 8, TPW), jnp.float32),
        mesh=_mesh,
        compiler_params=pltpu.CompilerParams(
            use_tc_tiling_on_sc=False, needs_layout_passes=False),
        scratch_types=[
            pltpu.VMEM((COLS, TPW), jnp.int32),
            pltpu.VMEM((4, TPW, 2 * D), jnp.float32),
            pltpu.VMEM((4, D // 8, 8, TPW), jnp.float32),
            pltpu.SemaphoreType.DMA((4,)),
            pltpu.SemaphoreType.DMA((4,)),
        ],
    )
    return k(xt, tab)


def kernel(x, table):
    xt = x.T
    tab = jnp.broadcast_to(table[:, None, :], (VOCAB, 2, D)).reshape(VOCAB, 2 * D)
    out6 = _embed(xt, tab)
    return out6.transpose(2, 4, 0, 1, 3).reshape(ROWS, COLS, D)


# speedup vs baseline: 1.0807x; 1.0807x over previous
import math

import jax
import jax.numpy as jnp
from jax import lax
from jax.experimental import pallas as pl
from jax.experimental.pallas import tpu as pltpu
from jax.experimental.pallas import tpu_sc as plsc

VOCAB = 1000000
D = 64
ROWS = 4096
COLS = 200
NC = 2
NS = 16
NW = NC * NS
TPW = ROWS // NW
L = 16
SCALE = math.sqrt(D)

_mesh = plsc.VectorSubcoreMesh(
    core_axis_name="c", subcore_axis_name="s", num_cores=NC, num_subcores=NS
)


def _body(x_hbm, tab_hbm, out_hbm, xq_v, in_v, out_v, gsem, ssem):
    wid = lax.axis_index("s") * NC + lax.axis_index("c")
    a0 = wid * TPW
    pltpu.sync_copy(x_hbm.at[:, pl.ds(a0, TPW)], xq_v)

    def start_gather(s, b):
        pltpu.async_copy(
            tab_hbm.at[xq_v.at[s]], in_v.at[b], gsem.at[b])

    def wait_gather(s, b):
        pltpu.make_async_copy(
            tab_hbm.at[xq_v.at[s]], in_v.at[b], gsem.at[b]).wait()

    def start_store(s, b):
        pltpu.async_copy(out_v.at[b], out_hbm.at[s, pl.ds(0, D // 8), wid], ssem.at[b])

    def wait_store(s, b):
        pltpu.make_async_copy(
            out_v.at[b], out_hbm.at[s, pl.ds(0, D // 8), wid], ssem.at[b]).wait()

    def transpose_add(s, b):
        lanes = lax.broadcasted_iota(jnp.int32, (L,), 0)

        @plsc.parallel_loop(0, (TPW // L) * (D // L), unroll=2)
        def _chunk(i):
            p = i >> 2
            c16 = (i & 3) * L
            rows = lanes + p * L
            for k in range(L):
                cc = ((lanes + k) & (L - 1)) + c16
                vals = plsc.load_gather(in_v.at[b], [rows, cc])
                plsc.store_scatter(
                    out_v.at[b], [cc >> 3, cc & 7, rows], vals + SCALE)


    for b in range(2):
        start_gather(b, b)

    def main(i, _):
        s0 = i * 4
        for j in range(4):
            s = s0 + j

            @pl.when(s >= 2)
            def _():
                start_store(s - 2, (j + 2) % 4)

            @pl.when(s + 2 < COLS)
            def _():
                start_gather(s + 2, (j + 2) % 4)

            wait_gather(s, j)

            @pl.when(s >= 4)
            def _():
                wait_store(s - 4, j)

            transpose_add(s, j)
        return 0

    lax.fori_loop(0, COLS // 4, main, 0)

    for s in range(COLS - 2, COLS):
        start_store(s, s % 4)
    for s in range(COLS - 4, COLS):
        wait_store(s, s % 4)


@jax.jit
def _embed(xt, tab):
    k = pl.kernel(
        _body,
        out_type=jax.ShapeDtypeStruct((COLS, D // 8, ROWS // TPW, 8, TPW), jnp.float32),
        mesh=_mesh,
        compiler_params=pltpu.CompilerParams(
            use_tc_tiling_on_sc=False, needs_layout_passes=False),
        scratch_types=[
            pltpu.VMEM((COLS, TPW), jnp.int32),
            pltpu.VMEM((4, TPW, 2 * D), jnp.float32),
            pltpu.VMEM((4, D // 8, 8, TPW), jnp.float32),
            pltpu.SemaphoreType.DMA((4,)),
            pltpu.SemaphoreType.DMA((4,)),
        ],
    )
    return k(xt, tab)


def kernel(x, table):
    xt = x.T
    tab = jnp.pad(table, ((0, 0), (0, D)))
    out6 = _embed(xt, tab)
    return out6.transpose(2, 4, 0, 1, 3).reshape(ROWS, COLS, D)
